# Initial kernel scaffold; baseline (speedup 1.0000x reference)
import jax
import jax.numpy as jnp
from jax.experimental import pallas as pl


def kernel(x, path_attr, node_attr, edge_attr, edge_index, ids, W_init, b_init, embed, W_self0, W_nbr0, W_edge0, gamma0, beta0, W_self1, W_nbr1, W_edge1, gamma1, beta1):
    raise NotImplementedError("write your pallas kernel here")


# speedup vs baseline: 2.4804x; 2.4804x over previous
import functools

import jax
import jax.numpy as jnp
from jax import lax
from jax.experimental import pallas as pl
from jax.experimental.pallas import tpu as pltpu
from jax.experimental.pallas import tpu_sc as plsc

NC, NS = 2, 16
NW = NC * NS
CH = 80


def _sc_mesh():
    return plsc.VectorSubcoreMesh(
        core_axis_name="c", subcore_axis_name="s",
        num_cores=NC, num_subcores=NS)


def _emb_gather(embed, ids):
    n, = ids.shape
    d = embed.shape[1]
    chunks = n // CH
    iters = (chunks + NW - 1) // NW

    @functools.partial(
        pl.kernel,
        out_type=jax.ShapeDtypeStruct((n, d), jnp.float32),
        mesh=_sc_mesh(),
        scratch_types=[
            pltpu.VMEM((CH,), jnp.int32),
            pltpu.VMEM((CH, d), jnp.float32),
            pltpu.SemaphoreType.DMA,
        ],
    )
    def k(emb_hbm, ids_hbm, out_hbm, idx_v, rows_v, sem):
        wid = lax.axis_index("s") * NC + lax.axis_index("c")

        def body(j, carry):
            c = wid + j * NW

            @pl.when(c < chunks)
            def _():
                base = c * CH
                pltpu.sync_copy(ids_hbm.at[pl.ds(base, CH)], idx_v)
                pltpu.async_copy(emb_hbm.at[idx_v], rows_v, sem).wait()
                pltpu.sync_copy(rows_v, out_hbm.at[pl.ds(base, CH)])

            return carry

        lax.fori_loop(0, iters, body, 0)

    return k(embed, ids)


def _edge_pass(h, msg, src, dst, zeros):
    n, d = h.shape
    e = src.shape[0]
    epw = e // NW
    chunks = epw // CH
    r_full = ((n + NS - 1) // NS + 7) // 8 * 8
    r_last = n - (NS - 1) * r_full
    assert 0 < r_last <= r_full

    @functools.partial(
        pl.kernel,
        out_type=jax.ShapeDtypeStruct((NC * n, d), jnp.float32),
        mesh=_sc_mesh(),
        scratch_types=[
            pltpu.VMEM((CH,), jnp.int32),
            pltpu.VMEM((CH,), jnp.int32),
            pltpu.VMEM((CH, d), jnp.float32),
            pltpu.VMEM((CH, d), jnp.float32),
            pltpu.VMEM_SHARED((n, d), jnp.float32),
            pltpu.SemaphoreType.DMA,
        ],
    )
    def k(h_hbm, msg_hbm, src_hbm, dst_hbm, zero_hbm, out_hbm,
          sidx, didx, gbuf, mbuf, agg, sem):
        cid = lax.axis_index("c")
        sid = lax.axis_index("s")
        wid = sid * NC + cid
        @pl.when(sid < NS - 1)
        def _():
            pltpu.sync_copy(zero_hbm, agg.at[pl.ds(sid * r_full, r_full)])

        @pl.when(sid == NS - 1)
        def _():
            pltpu.sync_copy(zero_hbm.at[pl.ds(0, r_last)],
                            agg.at[pl.ds(sid * r_full, r_last)])

        plsc.subcore_barrier()

        base = wid * epw

        def body(i, carry):
            eb = base + i * CH
            pltpu.sync_copy(src_hbm.at[pl.ds(eb, CH)], sidx)
            pltpu.sync_copy(dst_hbm.at[pl.ds(eb, CH)], didx)
            pltpu.async_copy(h_hbm.at[sidx], gbuf, sem).wait()
            pltpu.sync_copy(msg_hbm.at[pl.ds(eb, CH)], mbuf)
            pltpu.sync_copy(gbuf, agg.at[didx], add=True)
            pltpu.sync_copy(mbuf, agg.at[didx], add=True)
            return carry

        lax.fori_loop(0, chunks, body, 0)
        plsc.subcore_barrier()

        @pl.when(sid < NS - 1)
        def _():
            pltpu.sync_copy(agg.at[pl.ds(sid * r_full, r_full)],
                            out_hbm.at[pl.ds(cid * n + sid * r_full, r_full)])

        @pl.when(sid == NS - 1)
        def _():
            pltpu.sync_copy(agg.at[pl.ds(sid * r_full, r_last)],
                            out_hbm.at[pl.ds(cid * n + sid * r_full, r_last)])

    return k(h, msg, src, dst, zeros)


def _tc_init(x, W, b, emb):
    n, d = x.shape
    br = 400

    def body(x_ref, w_ref, b_ref, e_ref, o_ref):
        o_ref[...] = (jnp.dot(x_ref[...], w_ref[...],
                              preferred_element_type=jnp.float32)
                      + b_ref[...] + e_ref[...])

    return pl.pallas_call(
        body,
        grid=(n // br,),
        in_specs=[pl.BlockSpec((br, d), lambda i: (i, 0)),
                  pl.BlockSpec((d, d), lambda i: (0, 0)),
                  pl.BlockSpec((1, d), lambda i: (0, 0)),
                  pl.BlockSpec((br, d), lambda i: (i, 0))],
        out_specs=pl.BlockSpec((br, d), lambda i: (i, 0)),
        out_shape=jax.ShapeDtypeStruct((n, d), jnp.float32),
    )(x, W, b.reshape(1, d), emb)


def _tc_msgs(ea, W0, W1):
    e, de = ea.shape
    d = W0.shape[1]
    be = 2000

    def body(a_ref, w0_ref, w1_ref, o0_ref, o1_ref):
        a = a_ref[...]
        o0_ref[...] = jnp.maximum(
            jnp.dot(a, w0_ref[...], preferred_element_type=jnp.float32), 0.0)
        o1_ref[...] = jnp.maximum(
            jnp.dot(a, w1_ref[...], preferred_element_type=jnp.float32), 0.0)

    return pl.pallas_call(
        body,
        grid=(e // be,),
        in_specs=[pl.BlockSpec((be, de), lambda i: (i, 0)),
                  pl.BlockSpec((de, d), lambda i: (0, 0)),
                  pl.BlockSpec((de, d), lambda i: (0, 0))],
        out_specs=[pl.BlockSpec((be, d), lambda i: (i, 0)),
                   pl.BlockSpec((be, d), lambda i: (i, 0))],
        out_shape=[jax.ShapeDtypeStruct((e, d), jnp.float32),
                   jax.ShapeDtypeStruct((e, d), jnp.float32)],
    )(ea, W0, W1)


def _tc_update(a0, a1, h, W_n, W_s, g, b):
    n, d = h.shape
    br = 400

    def body(a0_ref, a1_ref, h_ref, wn_ref, ws_ref, g_ref, b_ref, o_ref):
        hn = (jnp.dot(a0_ref[...] + a1_ref[...], wn_ref[...],
                      preferred_element_type=jnp.float32)
              + jnp.dot(h_ref[...], ws_ref[...],
                        preferred_element_type=jnp.float32))
        mu = jnp.mean(hn, axis=-1, keepdims=True)
        var = jnp.mean((hn - mu) ** 2, axis=-1, keepdims=True)
        o_ref[...] = ((hn - mu) * lax.rsqrt(var + 1e-5) * g_ref[...]
                      + b_ref[...])

    return pl.pallas_call(
        body,
        grid=(n // br,),
        in_specs=[pl.BlockSpec((br, d), lambda i: (i, 0)),
                  pl.BlockSpec((br, d), lambda i: (i, 0)),
                  pl.BlockSpec((br, d), lambda i: (i, 0)),
                  pl.BlockSpec((d, d), lambda i: (0, 0)),
                  pl.BlockSpec((d, d), lambda i: (0, 0)),
                  pl.BlockSpec((1, d), lambda i: (0, 0)),
                  pl.BlockSpec((1, d), lambda i: (0, 0))],
        out_specs=pl.BlockSpec((br, d), lambda i: (i, 0)),
        out_shape=jax.ShapeDtypeStruct((n, d), jnp.float32),
    )(a0, a1, h, W_n, W_s, g.reshape(1, d), b.reshape(1, d))


def kernel(x, path_attr, node_attr, edge_attr, edge_index, ids,
           W_init, b_init, embed,
           W_self0, W_nbr0, W_edge0, gamma0, beta0,
           W_self1, W_nbr1, W_edge1, gamma1, beta1):
    n, d = x.shape
    ei = edge_index.astype(jnp.int32)
    src, dst = ei[0], ei[1]
    r_full = ((n + NS - 1) // NS + 7) // 8 * 8
    zeros = jnp.zeros((r_full, d), jnp.float32)

    emb = _emb_gather(embed, ids.astype(jnp.int32))
    h = _tc_init(x, W_init, b_init, emb)
    m0, m1 = _tc_msgs(edge_attr, W_edge0, W_edge1)

    for (msg, W_s, W_n, g, b) in ((m0, W_self0, W_nbr0, gamma0, beta0),
                                  (m1, W_self1, W_nbr1, gamma1, beta1)):
        aggp = _edge_pass(h, msg, src, dst, zeros)
        h = _tc_update(aggp[:n], aggp[n:], h, W_n, W_s, g, b)
    return h


# speedup vs baseline: 4.3541x; 1.7555x over previous
import functools

import jax
import jax.numpy as jnp
from jax import lax
from jax.experimental import pallas as pl
from jax.experimental.pallas import tpu as pltpu
from jax.experimental.pallas import tpu_sc as plsc

NC, NS = 2, 16
NW = NC * NS
CH = 80


def _sc_mesh():
    return plsc.VectorSubcoreMesh(
        core_axis_name="c", subcore_axis_name="s",
        num_cores=NC, num_subcores=NS)


def _emb_gather(embed, ids):
    n, = ids.shape
    d = embed.shape[1]
    chunks = n // CH
    iters = (chunks + NW - 1) // NW

    @functools.partial(
        pl.kernel,
        out_type=jax.ShapeDtypeStruct((n, d), jnp.float32),
        mesh=_sc_mesh(),
        scratch_types=[
            pltpu.VMEM((CH,), jnp.int32),
            pltpu.VMEM((CH, d), jnp.float32),
            pltpu.SemaphoreType.DMA,
        ],
    )
    def k(emb_hbm, ids_hbm, out_hbm, idx_v, rows_v, sem):
        wid = lax.axis_index("s") * NC + lax.axis_index("c")

        def body(j, carry):
            c = wid + j * NW

            @pl.when(c < chunks)
            def _():
                base = c * CH
                pltpu.sync_copy(ids_hbm.at[pl.ds(base, CH)], idx_v)
                pltpu.async_copy(emb_hbm.at[idx_v], rows_v, sem).wait()
                pltpu.sync_copy(rows_v, out_hbm.at[pl.ds(base, CH)])

            return carry

        lax.fori_loop(0, iters, body, 0)

    return k(embed, ids)


def _edge_pass(h, msg, src, dst, zeros):
    n, d = h.shape
    e = src.shape[0]
    epw = e // NW
    ce = 64
    chunks = epw // ce
    tail = epw - chunks * ce
    assert tail % 8 == 0 and chunks % 2 == 0
    r_full = ((n + NS - 1) // NS + 7) // 8 * 8
    r_last = n - (NS - 1) * r_full
    assert 0 < r_last <= r_full

    @functools.partial(
        pl.kernel,
        out_type=jax.ShapeDtypeStruct((NC * n, d), jnp.float32),
        mesh=_sc_mesh(),
        scratch_types=[
            pltpu.VMEM((epw,), jnp.int32),
            pltpu.VMEM((ce,), jnp.int32),
            pltpu.VMEM((ce,), jnp.int32),
            pltpu.VMEM((ce, d), jnp.float32),
            pltpu.VMEM((ce, d), jnp.float32),
            pltpu.VMEM((ce, d), jnp.float32),
            pltpu.VMEM((ce, d), jnp.float32),
            pltpu.VMEM((tail,), jnp.int32),
            pltpu.VMEM((tail, d), jnp.float32),
            pltpu.VMEM((tail, d), jnp.float32),
            pltpu.VMEM_SHARED((n, d), jnp.float32),
            pltpu.SemaphoreType.DMA,
            pltpu.SemaphoreType.DMA,
        ],
    )
    def k(h_hbm, msg_hbm, src_hbm, dst_hbm, zero_hbm, out_hbm,
          sidx, didx0, didx1, gb0, gb1, mb0, mb1, dtl, gtl, mtl,
          agg, sem0, sem1):
        cid = lax.axis_index("c")
        sid = lax.axis_index("s")
        wid = sid * NC + cid
        base = wid * epw
        @pl.when(sid < NS - 1)
        def _():
            pltpu.sync_copy(zero_hbm, agg.at[pl.ds(sid * r_full, r_full)])

        @pl.when(sid == NS - 1)
        def _():
            pltpu.sync_copy(zero_hbm.at[pl.ds(0, r_last)],
                            agg.at[pl.ds(sid * r_full, r_last)])

        pltpu.sync_copy(src_hbm.at[pl.ds(base, epw)], sidx)
        plsc.subcore_barrier()

        def issue(c, nrows, db, gb, mb, sem):
            eb = base + c * ce
            pltpu.async_copy(dst_hbm.at[pl.ds(eb, nrows)], db, sem)
            pltpu.async_copy(h_hbm.at[sidx.at[pl.ds(c * ce, nrows)]], gb, sem)
            pltpu.async_copy(msg_hbm.at[pl.ds(eb, nrows)], mb, sem)

        def drain(c, nrows, db, gb, mb, sem):
            eb = base + c * ce
            pltpu.make_async_copy(dst_hbm.at[pl.ds(eb, nrows)], db, sem).wait()
            pltpu.make_async_copy(
                h_hbm.at[sidx.at[pl.ds(c * ce, nrows)]], gb, sem).wait()
            pltpu.make_async_copy(msg_hbm.at[pl.ds(eb, nrows)], mb, sem).wait()

        def scat(db, gb, mb):
            pltpu.sync_copy(gb, agg.at[db], add=True)
            pltpu.sync_copy(mb, agg.at[db], add=True)

        issue(0, ce, didx0, gb0, mb0, sem0)
        issue(1, ce, didx1, gb1, mb1, sem1)

        def body(kk, carry):
            c0 = kk * 2
            drain(c0, ce, didx0, gb0, mb0, sem0)
            scat(didx0, gb0, mb0)

            @pl.when(c0 + 2 < chunks)
            def _():
                issue(c0 + 2, ce, didx0, gb0, mb0, sem0)

            drain(c0 + 1, ce, didx1, gb1, mb1, sem1)
            scat(didx1, gb1, mb1)

            @pl.when(c0 + 3 < chunks)
            def _():
                issue(c0 + 3, ce, didx1, gb1, mb1, sem1)

            return carry

        lax.fori_loop(0, chunks // 2, body, 0)
        if tail:
            issue(chunks, tail, dtl, gtl, mtl, sem0)
            drain(chunks, tail, dtl, gtl, mtl, sem0)
            scat(dtl, gtl, mtl)
        plsc.subcore_barrier()

        @pl.when(sid < NS - 1)
        def _():
            pltpu.sync_copy(agg.at[pl.ds(sid * r_full, r_full)],
                            out_hbm.at[pl.ds(cid * n + sid * r_full, r_full)])

        @pl.when(sid == NS - 1)
        def _():
            pltpu.sync_copy(agg.at[pl.ds(sid * r_full, r_last)],
                            out_hbm.at[pl.ds(cid * n + sid * r_full, r_last)])

    return k(h, msg, src, dst, zeros)


def _tc_init(x, W, b, emb):
    n, d = x.shape
    br = 400

    def body(x_ref, w_ref, b_ref, e_ref, o_ref):
        o_ref[...] = (jnp.dot(x_ref[...], w_ref[...],
                              preferred_element_type=jnp.float32)
                      + b_ref[...] + e_ref[...])

    return pl.pallas_call(
        body,
        grid=(n // br,),
        in_specs=[pl.BlockSpec((br, d), lambda i: (i, 0)),
                  pl.BlockSpec((d, d), lambda i: (0, 0)),
                  pl.BlockSpec((1, d), lambda i: (0, 0)),
                  pl.BlockSpec((br, d), lambda i: (i, 0))],
        out_specs=pl.BlockSpec((br, d), lambda i: (i, 0)),
        out_shape=jax.ShapeDtypeStruct((n, d), jnp.float32),
    )(x, W, b.reshape(1, d), emb)


def _tc_msgs(ea, W0):
    e, de = ea.shape
    d = W0.shape[1]
    be = 2000

    def body(a_ref, w0_ref, o0_ref):
        o0_ref[...] = jnp.maximum(
            jnp.dot(a_ref[...], w0_ref[...],
                    preferred_element_type=jnp.float32), 0.0)

    return pl.pallas_call(
        body,
        grid=(e // be,),
        in_specs=[pl.BlockSpec((be, de), lambda i: (i, 0)),
                  pl.BlockSpec((de, d), lambda i: (0, 0))],
        out_specs=pl.BlockSpec((be, d), lambda i: (i, 0)),
        out_shape=jax.ShapeDtypeStruct((e, d), jnp.float32),
    )(ea, W0)


def _tc_update(a0, a1, h, W_n, W_s, g, b):
    n, d = h.shape
    br = 400

    def body(a0_ref, a1_ref, h_ref, wn_ref, ws_ref, g_ref, b_ref, o_ref):
        hn = (jnp.dot(a0_ref[...] + a1_ref[...], wn_ref[...],
                      preferred_element_type=jnp.float32)
              + jnp.dot(h_ref[...], ws_ref[...],
                        preferred_element_type=jnp.float32))
        mu = jnp.mean(hn, axis=-1, keepdims=True)
        var = jnp.mean((hn - mu) ** 2, axis=-1, keepdims=True)
        o_ref[...] = ((hn - mu) * lax.rsqrt(var + 1e-5) * g_ref[...]
                      + b_ref[...])

    return pl.pallas_call(
        body,
        grid=(n // br,),
        in_specs=[pl.BlockSpec((br, d), lambda i: (i, 0)),
                  pl.BlockSpec((br, d), lambda i: (i, 0)),
                  pl.BlockSpec((br, d), lambda i: (i, 0)),
                  pl.BlockSpec((d, d), lambda i: (0, 0)),
                  pl.BlockSpec((d, d), lambda i: (0, 0)),
                  pl.BlockSpec((1, d), lambda i: (0, 0)),
                  pl.BlockSpec((1, d), lambda i: (0, 0))],
        out_specs=pl.BlockSpec((br, d), lambda i: (i, 0)),
        out_shape=jax.ShapeDtypeStruct((n, d), jnp.float32),
    )(a0, a1, h, W_n, W_s, g.reshape(1, d), b.reshape(1, d))


def kernel(x, path_attr, node_attr, edge_attr, edge_index, ids,
           W_init, b_init, embed,
           W_self0, W_nbr0, W_edge0, gamma0, beta0,
           W_self1, W_nbr1, W_edge1, gamma1, beta1):
    n, d = x.shape
    ei = edge_index.astype(jnp.int32)
    src, dst = ei[0], ei[1]
    r_full = ((n + NS - 1) // NS + 7) // 8 * 8
    zeros = jnp.zeros((r_full, d), jnp.float32)

    emb = _emb_gather(embed, ids.astype(jnp.int32))
    h = _tc_init(x, W_init, b_init, emb)
    m0 = _tc_msgs(edge_attr, W_edge0)
    m1 = _tc_msgs(edge_attr, W_edge1)

    for (msg, W_s, W_n, g, b) in ((m0, W_self0, W_nbr0, gamma0, beta0),
                                  (m1, W_self1, W_nbr1, gamma1, beta1)):
        aggp = _edge_pass(h, msg, src, dst, zeros)
        h = _tc_update(aggp[:n], aggp[n:], h, W_n, W_s, g, b)
    return h


# speedup vs baseline: 4.6560x; 1.0693x over previous
import functools

import jax
import jax.numpy as jnp
from jax import lax
from jax.experimental import pallas as pl
from jax.experimental.pallas import tpu as pltpu
from jax.experimental.pallas import tpu_sc as plsc

NC, NS = 2, 16
NW = NC * NS
CH = 80


def _sc_mesh():
    return plsc.VectorSubcoreMesh(
        core_axis_name="c", subcore_axis_name="s",
        num_cores=NC, num_subcores=NS)


def _emb_gather(embed, ids):
    n, = ids.shape
    d = embed.shape[1]
    chunks = n // CH
    iters = (chunks + NW - 1) // NW

    @functools.partial(
        pl.kernel,
        out_type=jax.ShapeDtypeStruct((n, d), jnp.float32),
        mesh=_sc_mesh(),
        scratch_types=[
            pltpu.VMEM((CH,), jnp.int32),
            pltpu.VMEM((CH, d), jnp.float32),
            pltpu.SemaphoreType.DMA,
        ],
    )
    def k(emb_hbm, ids_hbm, out_hbm, idx_v, rows_v, sem):
        wid = lax.axis_index("s") * NC + lax.axis_index("c")

        def body(j, carry):
            c = wid + j * NW

            @pl.when(c < chunks)
            def _():
                base = c * CH
                pltpu.sync_copy(ids_hbm.at[pl.ds(base, CH)], idx_v)
                pltpu.async_copy(emb_hbm.at[idx_v], rows_v, sem).wait()
                pltpu.sync_copy(rows_v, out_hbm.at[pl.ds(base, CH)])

            return carry

        lax.fori_loop(0, iters, body, 0)

    return k(embed, ids)


def _edge_pass(h, msg, src, dst, zeros):
    n, d = h.shape
    e = src.shape[0]
    epw = e // NW
    ce = 64
    chunks = epw // ce
    tail = epw - chunks * ce
    assert tail % 8 == 0 and chunks % 2 == 0
    r_full = ((n + NS - 1) // NS + 7) // 8 * 8
    r_last = n - (NS - 1) * r_full
    assert 0 < r_last <= r_full

    @functools.partial(
        pl.kernel,
        out_type=jax.ShapeDtypeStruct((NC * n, d), jnp.float32),
        mesh=_sc_mesh(),
        scratch_types=[
            pltpu.VMEM((epw,), jnp.int32),
            pltpu.VMEM((ce,), jnp.int32),
            pltpu.VMEM((ce,), jnp.int32),
            pltpu.VMEM((ce, d), jnp.float32),
            pltpu.VMEM((ce, d), jnp.float32),
            pltpu.VMEM((ce, d), jnp.float32),
            pltpu.VMEM((ce, d), jnp.float32),
            pltpu.VMEM((tail,), jnp.int32),
            pltpu.VMEM((tail, d), jnp.float32),
            pltpu.VMEM((tail, d), jnp.float32),
            pltpu.VMEM_SHARED((n, d), jnp.float32),
            pltpu.SemaphoreType.DMA,
            pltpu.SemaphoreType.DMA,
        ],
    )
    def k(h_hbm, msg_hbm, src_hbm, dst_hbm, zero_hbm, out_hbm,
          sidx, didx0, didx1, gb0, gb1, mb0, mb1, dtl, gtl, mtl,
          agg, sem0, sem1):
        cid = lax.axis_index("c")
        sid = lax.axis_index("s")
        wid = sid * NC + cid
        base = wid * epw
        @pl.when(sid < NS - 1)
        def _():
            pltpu.sync_copy(zero_hbm, agg.at[pl.ds(sid * r_full, r_full)])

        @pl.when(sid == NS - 1)
        def _():
            pltpu.sync_copy(zero_hbm.at[pl.ds(0, r_last)],
                            agg.at[pl.ds(sid * r_full, r_last)])

        pltpu.sync_copy(src_hbm.at[pl.ds(base, epw)], sidx)
        plsc.subcore_barrier()

        def issue(c, nrows, db, gb, mb, sem):
            eb = base + c * ce
            pltpu.async_copy(dst_hbm.at[pl.ds(eb, nrows)], db, sem)
            pltpu.async_copy(h_hbm.at[sidx.at[pl.ds(c * ce, nrows)]], gb, sem)
            pltpu.async_copy(msg_hbm.at[pl.ds(eb, nrows)], mb, sem)

        def drain(c, nrows, db, gb, mb, sem):
            eb = base + c * ce
            pltpu.make_async_copy(dst_hbm.at[pl.ds(eb, nrows)], db, sem).wait()
            pltpu.make_async_copy(
                h_hbm.at[sidx.at[pl.ds(c * ce, nrows)]], gb, sem).wait()
            pltpu.make_async_copy(msg_hbm.at[pl.ds(eb, nrows)], mb, sem).wait()

        def scat(db, gb, mb):
            pltpu.sync_copy(gb, agg.at[db], add=True)
            pltpu.sync_copy(mb, agg.at[db], add=True)

        issue(0, ce, didx0, gb0, mb0, sem0)
        issue(1, ce, didx1, gb1, mb1, sem1)

        def body(kk, carry):
            c0 = kk * 2
            drain(c0, ce, didx0, gb0, mb0, sem0)
            scat(didx0, gb0, mb0)

            @pl.when(c0 + 2 < chunks)
            def _():
                issue(c0 + 2, ce, didx0, gb0, mb0, sem0)

            drain(c0 + 1, ce, didx1, gb1, mb1, sem1)
            scat(didx1, gb1, mb1)

            @pl.when(c0 + 3 < chunks)
            def _():
                issue(c0 + 3, ce, didx1, gb1, mb1, sem1)

            return carry

        lax.fori_loop(0, chunks // 2, body, 0)
        if tail:
            issue(chunks, tail, dtl, gtl, mtl, sem0)
            drain(chunks, tail, dtl, gtl, mtl, sem0)
            scat(dtl, gtl, mtl)
        plsc.subcore_barrier()

        @pl.when(sid < NS - 1)
        def _():
            pltpu.sync_copy(agg.at[pl.ds(sid * r_full, r_full)],
                            out_hbm.at[pl.ds(cid * n + sid * r_full, r_full)])

        @pl.when(sid == NS - 1)
        def _():
            pltpu.sync_copy(agg.at[pl.ds(sid * r_full, r_last)],
                            out_hbm.at[pl.ds(cid * n + sid * r_full, r_last)])

    return k(h, msg, src, dst, zeros)


def _tc_init(x, W, b, emb):
    n, d = x.shape
    br = 400

    def body(x_ref, w_ref, b_ref, e_ref, o_ref):
        o_ref[...] = (jnp.dot(x_ref[...], w_ref[...],
                              preferred_element_type=jnp.float32)
                      + b_ref[...] + e_ref[...])

    return pl.pallas_call(
        body,
        grid=(n // br,),
        in_specs=[pl.BlockSpec((br, d), lambda i: (i, 0)),
                  pl.BlockSpec((d, d), lambda i: (0, 0)),
                  pl.BlockSpec((1, d), lambda i: (0, 0)),
                  pl.BlockSpec((br, d), lambda i: (i, 0))],
        out_specs=pl.BlockSpec((br, d), lambda i: (i, 0)),
        out_shape=jax.ShapeDtypeStruct((n, d), jnp.float32),
    )(x, W, b.reshape(1, d), emb)


def _tc_msgs(ea, W0):
    e, de = ea.shape
    d = W0.shape[1]
    be = 8000

    def body(a_ref, w0_ref, o0_ref):
        o0_ref[...] = jnp.maximum(
            jnp.dot(a_ref[...], w0_ref[...],
                    preferred_element_type=jnp.float32), 0.0)

    return pl.pallas_call(
        body,
        grid=(e // be,),
        in_specs=[pl.BlockSpec((be, de), lambda i: (i, 0)),
                  pl.BlockSpec((de, d), lambda i: (0, 0))],
        out_specs=pl.BlockSpec((be, d), lambda i: (i, 0)),
        out_shape=jax.ShapeDtypeStruct((e, d), jnp.float32),
    )(ea, W0)


def _tc_update(aggp, h, W_n, W_s, g, b):
    n, d = h.shape
    br = 400
    nb = n // br

    def body(a0_ref, a1_ref, h_ref, wn_ref, ws_ref, g_ref, b_ref, o_ref):
        hn = (jnp.dot(a0_ref[...] + a1_ref[...], wn_ref[...],
                      preferred_element_type=jnp.float32)
              + jnp.dot(h_ref[...], ws_ref[...],
                        preferred_element_type=jnp.float32))
        mu = jnp.mean(hn, axis=-1, keepdims=True)
        var = jnp.mean((hn - mu) ** 2, axis=-1, keepdims=True)
        o_ref[...] = ((hn - mu) * lax.rsqrt(var + 1e-5) * g_ref[...]
                      + b_ref[...])

    return pl.pallas_call(
        body,
        grid=(n // br,),
        in_specs=[pl.BlockSpec((br, d), lambda i: (i, 0)),
                  pl.BlockSpec((br, d), lambda i: (nb + i, 0)),
                  pl.BlockSpec((br, d), lambda i: (i, 0)),
                  pl.BlockSpec((d, d), lambda i: (0, 0)),
                  pl.BlockSpec((d, d), lambda i: (0, 0)),
                  pl.BlockSpec((1, d), lambda i: (0, 0)),
                  pl.BlockSpec((1, d), lambda i: (0, 0))],
        out_specs=pl.BlockSpec((br, d), lambda i: (i, 0)),
        out_shape=jax.ShapeDtypeStruct((n, d), jnp.float32),
    )(aggp, aggp, h, W_n, W_s, g.reshape(1, d), b.reshape(1, d))


def kernel(x, path_attr, node_attr, edge_attr, edge_index, ids,
           W_init, b_init, embed,
           W_self0, W_nbr0, W_edge0, gamma0, beta0,
           W_self1, W_nbr1, W_edge1, gamma1, beta1):
    n, d = x.shape
    ei = edge_index.astype(jnp.int32)
    src, dst = ei[0], ei[1]
    r_full = ((n + NS - 1) // NS + 7) // 8 * 8
    zeros = jnp.zeros((r_full, d), jnp.float32)

    emb = _emb_gather(embed, ids.astype(jnp.int32))
    h = _tc_init(x, W_init, b_init, emb)
    m0 = _tc_msgs(edge_attr, W_edge0)
    m1 = _tc_msgs(edge_attr, W_edge1)

    for (msg, W_s, W_n, g, b) in ((m0, W_self0, W_nbr0, gamma0, beta0),
                                  (m1, W_self1, W_nbr1, gamma1, beta1)):
        aggp = _edge_pass(h, msg, src, dst, zeros)
        h = _tc_update(aggp, h, W_n, W_s, g, b)
    return h
